# Initial kernel scaffold; baseline (speedup 1.0000x reference)
import jax
import jax.numpy as jnp
from jax.experimental import pallas as pl


def kernel(x, edge_index, batch, target, W1, b1, W2, b2, Wg, bg, Wc, bc, Wxt, bxt, Wf, bf, Wo, bo):
    raise NotImplementedError("write your pallas kernel here")


# speedup vs baseline: 12.6011x; 12.6011x over previous
import functools

import jax
import jax.numpy as jnp
from jax import lax
from jax.experimental import pallas as pl
from jax.experimental.pallas import tpu as pltpu
from jax.experimental.pallas import tpu_sc as plsc

_CH = 128
_NW = 32


def _sc_mesh():
    return plsc.VectorSubcoreMesh(core_axis_name="c", subcore_axis_name="s")


def _make_deg_kernel(n, nchunk):
    niter = (nchunk + _NW - 1) // _NW
    span = ((n // 16 + 15) // 16) * 16
    last = n - 15 * span

    @functools.partial(
        pl.kernel,
        out_type=jax.ShapeDtypeStruct((2 * n,), jnp.float32),
        mesh=_sc_mesh(),
        scratch_types=[
            pltpu.VMEM((1, _CH), jnp.int32),
            pltpu.VMEM((_CH,), jnp.float32),
            pltpu.VMEM((span,), jnp.float32),
            pltpu.VMEM_SHARED((n,), jnp.float32),
        ],
    )
    def deg_kernel(dst_hbm, out_hbm, idx_v, ones_v, stage_v, acc_sh):
        c = lax.axis_index("c")
        s = lax.axis_index("s")
        wid = s * 2 + c
        for i in range(_CH // 16):
            ones_v[pl.ds(i * 16, 16)] = jnp.ones((16,), jnp.float32)

        def zbody(i, carry):
            stage_v[pl.ds(i * 16, 16)] = jnp.zeros((16,), jnp.float32)
            return carry

        lax.fori_loop(0, span // 16, zbody, 0)
        @pl.when(s < 15)
        def _():
            pltpu.sync_copy(stage_v, acc_sh.at[pl.ds(s * span, span)])

        @pl.when(s == 15)
        def _():
            pltpu.sync_copy(stage_v.at[pl.ds(0, last)],
                            acc_sh.at[pl.ds(15 * span, last)])

        plsc.subcore_barrier()

        def body(j, carry):
            cidx = wid + _NW * j

            @pl.when(cidx < nchunk)
            def _():
                pltpu.sync_copy(dst_hbm.at[pl.ds(cidx, 1)], idx_v)
                pltpu.sync_copy(ones_v, acc_sh.at[idx_v.at[0]], add=True)

            return carry

        lax.fori_loop(0, niter, body, 0)
        plsc.subcore_barrier()

        @pl.when(s < 15)
        def _():
            pltpu.sync_copy(acc_sh.at[pl.ds(s * span, span)], stage_v)
            pltpu.sync_copy(stage_v, out_hbm.at[pl.ds(c * n + s * span, span)])

        @pl.when(s == 15)
        def _():
            pltpu.sync_copy(acc_sh.at[pl.ds(15 * span, last)],
                            stage_v.at[pl.ds(0, last)])
            pltpu.sync_copy(stage_v.at[pl.ds(0, last)],
                            out_hbm.at[pl.ds(c * n + 15 * span, last)])

    return deg_kernel


def _make_scatter_col_kernel(n, d, nchunk):
    niter = (nchunk + 15) // 16
    span = ((n // 16 + 7) // 8) * 8
    last = n - 15 * span

    @functools.partial(
        pl.kernel,
        out_type=[jax.ShapeDtypeStruct((n, d), jnp.float32),
                  jax.ShapeDtypeStruct((n, d), jnp.float32)],
        mesh=_sc_mesh(),
        scratch_types=[
            pltpu.VMEM((_CH,), jnp.int32),
            pltpu.VMEM((1, _CH), jnp.int32),
            pltpu.VMEM((_CH, d), jnp.float32),
            pltpu.VMEM((span, d), jnp.float32),
            pltpu.VMEM_SHARED((n, d), jnp.float32),
            pltpu.SemaphoreType.DMA,
        ],
        compiler_params=pltpu.CompilerParams(use_tc_tiling_on_sc=False),
    )
    def scat_kernel(ua_hbm, ub_hbm, src_hbm, dst_hbm, outa_hbm, outb_hbm,
                    sidx_v, didx_v, rows_v, stage_v, acc_sh, sem):
        c = lax.axis_index("c")
        s = lax.axis_index("s")

        def run(u_hbm, out_hbm):
            @pl.when(s < 15)
            def _():
                rb = s * span
                pltpu.sync_copy(u_hbm.at[pl.ds(rb, span)], stage_v)
                pltpu.sync_copy(stage_v, acc_sh.at[pl.ds(rb, span)])

            @pl.when(s == 15)
            def _():
                pltpu.sync_copy(u_hbm.at[pl.ds(15 * span, last)],
                                stage_v.at[pl.ds(0, last)])
                pltpu.sync_copy(stage_v.at[pl.ds(0, last)],
                                acc_sh.at[pl.ds(15 * span, last)])

            plsc.subcore_barrier()

            def body(j, carry):
                cidx = s + 16 * j

                @pl.when(cidx < nchunk)
                def _():
                    pltpu.sync_copy(src_hbm.at[pl.ds(cidx * _CH, _CH)], sidx_v)
                    pltpu.sync_copy(dst_hbm.at[pl.ds(cidx, 1)], didx_v)
                    pltpu.async_copy(u_hbm.at[sidx_v], rows_v, sem).wait()
                    pltpu.sync_copy(rows_v, acc_sh.at[didx_v.at[0]], add=True)

                return carry

            lax.fori_loop(0, niter, body, 0)
            plsc.subcore_barrier()

            @pl.when(s < 15)
            def _():
                rb = s * span
                pltpu.sync_copy(acc_sh.at[pl.ds(rb, span)], stage_v)
                pltpu.sync_copy(stage_v, out_hbm.at[pl.ds(rb, span)])

            @pl.when(s == 15)
            def _():
                pltpu.sync_copy(acc_sh.at[pl.ds(15 * span, last)],
                                stage_v.at[pl.ds(0, last)])
                pltpu.sync_copy(stage_v.at[pl.ds(0, last)],
                                out_hbm.at[pl.ds(15 * span, last)])

        @pl.when(c == 0)
        def _():
            run(ua_hbm, outa_hbm)

        @pl.when(c == 1)
        def _():
            run(ub_hbm, outb_hbm)

    return scat_kernel


def _make_scatter_kernel(n, d, nchunk):
    niter = (nchunk + _NW - 1) // _NW
    span = ((n // 16 + 7) // 8) * 8
    last = n - 15 * span

    @functools.partial(
        pl.kernel,
        out_type=jax.ShapeDtypeStruct((2 * n, d), jnp.float32),
        mesh=_sc_mesh(),
        scratch_types=[
            pltpu.VMEM((_CH,), jnp.int32),
            pltpu.VMEM((1, _CH), jnp.int32),
            pltpu.VMEM((_CH, d), jnp.float32),
            pltpu.VMEM((span, d), jnp.float32),
            pltpu.VMEM_SHARED((n, d), jnp.float32),
            pltpu.SemaphoreType.DMA,
        ],
        compiler_params=pltpu.CompilerParams(
            use_tc_tiling_on_sc=(d % 128 == 0)),
    )
    def scat_kernel(u_hbm, src_hbm, dst_hbm, out_hbm, sidx_v, didx_v, rows_v,
                    stage_v, acc_sh, sem):
        c = lax.axis_index("c")
        s = lax.axis_index("s")
        wid = s * 2 + c
        @pl.when(s < 15)
        def _():
            rb = s * span
            pltpu.sync_copy(u_hbm.at[pl.ds(rb, span)], stage_v)
            pltpu.sync_copy(stage_v, acc_sh.at[pl.ds(rb, span)])

        @pl.when(s == 15)
        def _():
            pltpu.sync_copy(u_hbm.at[pl.ds(15 * span, last)],
                            stage_v.at[pl.ds(0, last)])
            pltpu.sync_copy(stage_v.at[pl.ds(0, last)],
                            acc_sh.at[pl.ds(15 * span, last)])

        plsc.subcore_barrier()

        def body(j, carry):
            cidx = wid + _NW * j

            @pl.when(cidx < nchunk)
            def _():
                pltpu.sync_copy(src_hbm.at[pl.ds(cidx * _CH, _CH)], sidx_v)
                pltpu.sync_copy(dst_hbm.at[pl.ds(cidx, 1)], didx_v)
                pltpu.async_copy(u_hbm.at[sidx_v], rows_v, sem).wait()
                pltpu.sync_copy(rows_v, acc_sh.at[didx_v.at[0]], add=True)

            return carry

        lax.fori_loop(0, niter, body, 0)
        plsc.subcore_barrier()

        @pl.when(s < 15)
        def _():
            rb = s * span
            pltpu.sync_copy(acc_sh.at[pl.ds(rb, span)], stage_v)
            pltpu.sync_copy(stage_v, out_hbm.at[pl.ds(c * n + rb, span)])

        @pl.when(s == 15)
        def _():
            pltpu.sync_copy(acc_sh.at[pl.ds(15 * span, last)],
                            stage_v.at[pl.ds(0, last)])
            pltpu.sync_copy(stage_v.at[pl.ds(0, last)],
                            out_hbm.at[pl.ds(c * n + 15 * span, last)])

    return scat_kernel


def _prep_body(deg0, deg1, x, w1, dinv_o, u1_o):
    deg = deg0[...] + deg1[...] + 1.0
    dinv = lax.rsqrt(deg)
    dinv_o[...] = dinv
    u1_o[...] = dinv * jnp.dot(x[...], w1[...],
                               preferred_element_type=jnp.float32)


def _mid_body(acc0, acc1, u1, dinv, b1, w2a, w2b, u2a_o, u2b_o):
    di = dinv[...]
    h1 = jnp.maximum(di * (acc0[...] + acc1[...] - u1[...]) + b1[...], 0.0)
    u2a_o[...] = di * jnp.dot(h1, w2a[...], preferred_element_type=jnp.float32)
    u2b_o[...] = di * jnp.dot(h1, w2b[...], preferred_element_type=jnp.float32)


def _segmax_body(acc2a, acc2b, dinv, b2, batch, g0_o, *, num_graphs):
    i = pl.program_id(0)
    neg = jnp.float32(-jnp.inf)

    @pl.when(i == 0)
    def _():
        g0_o[...] = jnp.full(g0_o.shape, neg, jnp.float32)

    acc = jnp.concatenate([acc2a[...], acc2b[...]], axis=1)
    h2 = jnp.maximum(dinv[...] * acc + b2[...], 0.0)
    bb = batch[...]
    g_lo = jnp.min(bb)
    g_hi = jnp.max(bb)
    giota = lax.broadcasted_iota(jnp.int32, (num_graphs, 1), 0)

    def body(g, m_acc):
        m = jnp.max(jnp.where(bb == g, h2, neg), axis=0, keepdims=True)
        return jnp.where(giota == g, jnp.maximum(m_acc, m), m_acc)

    g0_o[...] = lax.fori_loop(g_lo, g_hi + 1, body, g0_o[...])


def _tail_body(g0, tgt, wg, bg, wc2, bc, wxt, bxt, wf, bf, wo, bo, out_o,
               *, bg_rows, seq, ctx, h):
    gv = jnp.dot(g0[...], wg[...], preferred_element_type=jnp.float32) + bg[...]
    t = tgt[...].reshape(bg_rows * seq, ctx)
    z = jnp.dot(t, wc2[...], preferred_element_type=jnp.float32)
    z = z.reshape(bg_rows, seq, 3 * h)
    y = (z[:, 0:seq - 2, 0:h] + z[:, 1:seq - 1, h:2 * h]
         + z[:, 2:seq, 2 * h:3 * h]) + bc[...]
    y = jnp.maximum(jnp.max(y, axis=1), 0.0)
    tv = jnp.dot(y, wxt[...], preferred_element_type=jnp.float32) + bxt[...]
    xc = jnp.concatenate([gv, tv], axis=1)
    f = jnp.maximum(
        jnp.dot(xc, wf[...], preferred_element_type=jnp.float32) + bf[...], 0.0)
    out_o[...] = jnp.dot(f, wo[...],
                         preferred_element_type=jnp.float32) + bo[...]


def kernel(x, edge_index, batch, target, W1, b1, W2, b2, Wg, bg, Wc, bc,
           Wxt, bxt, Wf, bf, Wo, bo):
    n, d_in = x.shape
    e = edge_index.shape[1]
    g, seq, ctx = target.shape
    h = W1.shape[1]
    h2 = W2.shape[1]
    nout = Wo.shape[1]
    nchunk = e // _CH

    src = edge_index[0]
    dst2d = edge_index[1].reshape(nchunk, _CH)

    deg2 = _make_deg_kernel(n, nchunk)(dst2d)
    deg2 = deg2.reshape(2 * n, 1)

    bn = 2000
    nblk = n // bn
    dinv, u1 = pl.pallas_call(
        _prep_body,
        grid=(nblk,),
        in_specs=[
            pl.BlockSpec((bn, 1), lambda i: (i, 0)),
            pl.BlockSpec((bn, 1), lambda i: (i + nblk, 0)),
            pl.BlockSpec((bn, d_in), lambda i: (i, 0)),
            pl.BlockSpec((d_in, h), lambda i: (0, 0)),
        ],
        out_specs=[
            pl.BlockSpec((bn, 1), lambda i: (i, 0)),
            pl.BlockSpec((bn, h), lambda i: (i, 0)),
        ],
        out_shape=[
            jax.ShapeDtypeStruct((n, 1), jnp.float32),
            jax.ShapeDtypeStruct((n, h), jnp.float32),
        ],
    )(deg2, deg2, x, W1)

    acc1 = _make_scatter_kernel(n, h, nchunk)(u1, src, dst2d)

    hh = h2 // 2
    u2a, u2b = pl.pallas_call(
        _mid_body,
        grid=(nblk,),
        in_specs=[
            pl.BlockSpec((bn, h), lambda i: (i, 0)),
            pl.BlockSpec((bn, h), lambda i: (i + nblk, 0)),
            pl.BlockSpec((bn, h), lambda i: (i, 0)),
            pl.BlockSpec((bn, 1), lambda i: (i, 0)),
            pl.BlockSpec((1, h), lambda i: (0, 0)),
            pl.BlockSpec((h, hh), lambda i: (0, 0)),
            pl.BlockSpec((h, hh), lambda i: (0, 0)),
        ],
        out_specs=[
            pl.BlockSpec((bn, hh), lambda i: (i, 0)),
            pl.BlockSpec((bn, hh), lambda i: (i, 0)),
        ],
        out_shape=[
            jax.ShapeDtypeStruct((n, hh), jnp.float32),
            jax.ShapeDtypeStruct((n, hh), jnp.float32),
        ],
    )(acc1, acc1, u1, dinv, b1.reshape(1, h), W2[:, :hh], W2[:, hh:])

    acc2a, acc2b = _make_scatter_col_kernel(n, hh, nchunk)(u2a, u2b, src,
                                                           dst2d)

    g0 = pl.pallas_call(
        functools.partial(_segmax_body, num_graphs=g),
        grid=(nblk,),
        in_specs=[
            pl.BlockSpec((bn, hh), lambda i: (i, 0)),
            pl.BlockSpec((bn, hh), lambda i: (i, 0)),
            pl.BlockSpec((bn, 1), lambda i: (i, 0)),
            pl.BlockSpec((1, h2), lambda i: (0, 0)),
            pl.BlockSpec((bn, 1), lambda i: (i, 0)),
        ],
        out_specs=pl.BlockSpec((g, h2), lambda i: (0, 0)),
        out_shape=jax.ShapeDtypeStruct((g, h2), jnp.float32),
    )(acc2a, acc2b, dinv, b2.reshape(1, h2), batch.reshape(n, 1))

    wc2 = jnp.transpose(Wc, (1, 2, 0)).reshape(ctx, 3 * h)
    wo_pad = jnp.zeros((h, 128), jnp.float32).at[:, :nout].set(Wo)
    bo_pad = jnp.zeros((1, 128), jnp.float32).at[:, :nout].set(bo)
    bg_rows = 8
    gblk = g // bg_rows
    out_pad = pl.pallas_call(
        functools.partial(_tail_body, bg_rows=bg_rows, seq=seq, ctx=ctx, h=h),
        grid=(gblk,),
        in_specs=[
            pl.BlockSpec((bg_rows, h2), lambda i: (i, 0)),
            pl.BlockSpec((bg_rows, seq, ctx), lambda i: (i, 0, 0)),
            pl.BlockSpec((h2, h), lambda i: (0, 0)),
            pl.BlockSpec((1, h), lambda i: (0, 0)),
            pl.BlockSpec((ctx, 3 * h), lambda i: (0, 0)),
            pl.BlockSpec((1, 1, h), lambda i: (0, 0, 0)),
            pl.BlockSpec((h, h), lambda i: (0, 0)),
            pl.BlockSpec((1, h), lambda i: (0, 0)),
            pl.BlockSpec((h2, h), lambda i: (0, 0)),
            pl.BlockSpec((1, h), lambda i: (0, 0)),
            pl.BlockSpec((h, 128), lambda i: (0, 0)),
            pl.BlockSpec((1, 128), lambda i: (0, 0)),
        ],
        out_specs=pl.BlockSpec((bg_rows, 128), lambda i: (i, 0)),
        out_shape=jax.ShapeDtypeStruct((g, 128), jnp.float32),
    )(g0, target, Wg, bg.reshape(1, h), wc2, bc.reshape(1, 1, h), Wxt,
      bxt.reshape(1, h), Wf, bf.reshape(1, h), wo_pad, bo_pad)

    return out_pad[:, :nout]


# speedup vs baseline: 18.3737x; 1.4581x over previous
import functools

import jax
import jax.numpy as jnp
from jax import lax
from jax.experimental import pallas as pl
from jax.experimental.pallas import tpu as pltpu
from jax.experimental.pallas import tpu_sc as plsc

_CH = 128
_NW = 32
_IB = 4
_PAD = 16


def _sc_mesh():
    return plsc.VectorSubcoreMesh(core_axis_name="c", subcore_axis_name="s")


def _make_deg_kernel(n, nchunkp):
    cpw = nchunkp // _NW
    nbatch = cpw // _IB
    span = ((n // 16 + 15) // 16) * 16
    last = n - 15 * span

    @functools.partial(
        pl.kernel,
        out_type=jax.ShapeDtypeStruct((2 * n,), jnp.float32),
        mesh=_sc_mesh(),
        scratch_types=[
            pltpu.VMEM((_IB, _CH), jnp.int32),
            pltpu.VMEM((_CH,), jnp.float32),
            pltpu.VMEM((span,), jnp.float32),
            pltpu.VMEM_SHARED((n + _PAD,), jnp.float32),
        ],
    )
    def deg_kernel(dst_hbm, out_hbm, idx_v, ones_v, stage_v, acc_sh):
        c = lax.axis_index("c")
        s = lax.axis_index("s")
        wid = s * 2 + c
        for i in range(_CH // 16):
            ones_v[pl.ds(i * 16, 16)] = jnp.ones((16,), jnp.float32)

        def zbody(i, carry):
            stage_v[pl.ds(i * 16, 16)] = jnp.zeros((16,), jnp.float32)
            return carry

        lax.fori_loop(0, span // 16, zbody, 0)
        @pl.when(s < 15)
        def _():
            pltpu.sync_copy(stage_v, acc_sh.at[pl.ds(s * span, span)])

        @pl.when(s == 15)
        def _():
            pltpu.sync_copy(stage_v.at[pl.ds(0, last + _PAD)],
                            acc_sh.at[pl.ds(15 * span, last + _PAD)])

        plsc.subcore_barrier()
        first = wid * cpw

        def body(b, carry):
            cbase = first + b * _IB
            pltpu.sync_copy(dst_hbm.at[pl.ds(cbase, _IB)], idx_v)
            for k in range(_IB):
                pltpu.sync_copy(ones_v, acc_sh.at[idx_v.at[k]], add=True)
            return carry

        lax.fori_loop(0, nbatch, body, 0)
        plsc.subcore_barrier()

        @pl.when(s < 15)
        def _():
            pltpu.sync_copy(acc_sh.at[pl.ds(s * span, span)], stage_v)
            pltpu.sync_copy(stage_v, out_hbm.at[pl.ds(c * n + s * span, span)])

        @pl.when(s == 15)
        def _():
            pltpu.sync_copy(acc_sh.at[pl.ds(15 * span, last)],
                            stage_v.at[pl.ds(0, last)])
            pltpu.sync_copy(stage_v.at[pl.ds(0, last)],
                            out_hbm.at[pl.ds(c * n + 15 * span, last)])

    return deg_kernel


def _make_scatter_col_kernel(n, d, nchunkp):
    cpt = nchunkp // 16
    nbatch = cpt // _IB
    span = ((n // 16 + 7) // 8) * 8
    last = n - 15 * span

    @functools.partial(
        pl.kernel,
        out_type=[jax.ShapeDtypeStruct((n, d), jnp.float32),
                  jax.ShapeDtypeStruct((n, d), jnp.float32)],
        mesh=_sc_mesh(),
        scratch_types=[
            pltpu.VMEM((_IB * _CH,), jnp.int32),
            pltpu.VMEM((_IB, _CH), jnp.int32),
            pltpu.VMEM((_IB * _CH, d), jnp.float32),
            pltpu.VMEM((span, d), jnp.float32),
            pltpu.VMEM_SHARED((n + _PAD, d), jnp.float32),
            pltpu.SemaphoreType.DMA,
        ],
        compiler_params=pltpu.CompilerParams(use_tc_tiling_on_sc=False),
    )
    def scat_kernel(ua_hbm, ub_hbm, src_hbm, dst_hbm, outa_hbm, outb_hbm,
                    sidx_v, didx_v, rows_v, stage_v, acc_sh, sem):
        c = lax.axis_index("c")
        s = lax.axis_index("s")

        def run(u_hbm, out_hbm):
            @pl.when(s < 15)
            def _():
                rb = s * span
                pltpu.sync_copy(u_hbm.at[pl.ds(rb, span)], stage_v)
                pltpu.sync_copy(stage_v, acc_sh.at[pl.ds(rb, span)])

            @pl.when(s == 15)
            def _():
                pltpu.sync_copy(u_hbm.at[pl.ds(15 * span, last)],
                                stage_v.at[pl.ds(0, last)])
                pltpu.sync_copy(stage_v.at[pl.ds(0, last)],
                                acc_sh.at[pl.ds(15 * span, last)])

            plsc.subcore_barrier()
            first = s * cpt

            def body(b, carry):
                cbase = first + b * _IB
                pltpu.sync_copy(src_hbm.at[pl.ds(cbase * _CH, _IB * _CH)],
                                sidx_v)
                pltpu.sync_copy(dst_hbm.at[pl.ds(cbase, _IB)], didx_v)
                cps = [
                    pltpu.async_copy(
                        u_hbm.at[sidx_v.at[pl.ds(k * _CH, _CH)]],
                        rows_v.at[pl.ds(k * _CH, _CH)], sem)
                    for k in range(_IB)
                ]
                for cp in cps:
                    cp.wait()
                for k in range(_IB):
                    pltpu.sync_copy(rows_v.at[pl.ds(k * _CH, _CH)],
                                    acc_sh.at[didx_v.at[k]], add=True)
                return carry

            lax.fori_loop(0, nbatch, body, 0)
            plsc.subcore_barrier()

            @pl.when(s < 15)
            def _():
                rb = s * span
                pltpu.sync_copy(acc_sh.at[pl.ds(rb, span)], stage_v)
                pltpu.sync_copy(stage_v, out_hbm.at[pl.ds(rb, span)])

            @pl.when(s == 15)
            def _():
                pltpu.sync_copy(acc_sh.at[pl.ds(15 * span, last)],
                                stage_v.at[pl.ds(0, last)])
                pltpu.sync_copy(stage_v.at[pl.ds(0, last)],
                                out_hbm.at[pl.ds(15 * span, last)])

        @pl.when(c == 0)
        def _():
            run(ua_hbm, outa_hbm)

        @pl.when(c == 1)
        def _():
            run(ub_hbm, outb_hbm)

    return scat_kernel


def _prep_body(deg0, deg1, x, w1a, w1b, dinv_o, u1a_o, u1b_o):
    deg = deg0[...] + deg1[...] + 1.0
    dinv = lax.rsqrt(deg)
    dinv_o[...] = dinv
    u1a_o[...] = dinv * jnp.dot(x[...], w1a[...],
                                preferred_element_type=jnp.float32)
    u1b_o[...] = dinv * jnp.dot(x[...], w1b[...],
                                preferred_element_type=jnp.float32)


def _mid_body(acc1a, acc1b, dinv, b1, w2a, w2b, u2a_o, u2b_o):
    di = dinv[...]
    acc = jnp.concatenate([acc1a[...], acc1b[...]], axis=1)
    h1 = jnp.maximum(di * acc + b1[...], 0.0)
    u2a_o[...] = di * jnp.dot(h1, w2a[...], preferred_element_type=jnp.float32)
    u2b_o[...] = di * jnp.dot(h1, w2b[...], preferred_element_type=jnp.float32)


def _segmax_body(acc2a, acc2b, dinv, b2, batch, g0_o, *, num_graphs):
    i = pl.program_id(0)
    neg = jnp.float32(-jnp.inf)

    @pl.when(i == 0)
    def _():
        g0_o[...] = jnp.full(g0_o.shape, neg, jnp.float32)

    acc = jnp.concatenate([acc2a[...], acc2b[...]], axis=1)
    h2 = jnp.maximum(dinv[...] * acc + b2[...], 0.0)
    bb = batch[...]
    g_lo = jnp.min(bb)
    g_hi = jnp.max(bb)
    giota = lax.broadcasted_iota(jnp.int32, (num_graphs, 1), 0)

    def body(g, m_acc):
        m = jnp.max(jnp.where(bb == g, h2, neg), axis=0, keepdims=True)
        return jnp.where(giota == g, jnp.maximum(m_acc, m), m_acc)

    g0_o[...] = lax.fori_loop(g_lo, g_hi + 1, body, g0_o[...])


def _tail_body(g0, tgt, wg, bg, wc2, bc, wxt, bxt, wf, bf, wo, bo, out_o,
               *, bg_rows, seq, ctx, h):
    gv = jnp.dot(g0[...], wg[...], preferred_element_type=jnp.float32) + bg[...]
    t = tgt[...].reshape(bg_rows * seq, ctx)
    z = jnp.dot(t, wc2[...], preferred_element_type=jnp.float32)
    z = z.reshape(bg_rows, seq, 3 * h)
    y = (z[:, 0:seq - 2, 0:h] + z[:, 1:seq - 1, h:2 * h]
         + z[:, 2:seq, 2 * h:3 * h]) + bc[...]
    y = jnp.maximum(jnp.max(y, axis=1), 0.0)
    tv = jnp.dot(y, wxt[...], preferred_element_type=jnp.float32) + bxt[...]
    xc = jnp.concatenate([gv, tv], axis=1)
    f = jnp.maximum(
        jnp.dot(xc, wf[...], preferred_element_type=jnp.float32) + bf[...], 0.0)
    out_o[...] = jnp.dot(f, wo[...],
                         preferred_element_type=jnp.float32) + bo[...]


def kernel(x, edge_index, batch, target, W1, b1, W2, b2, Wg, bg, Wc, bc,
           Wxt, bxt, Wf, bf, Wo, bo):
    n, d_in = x.shape
    e = edge_index.shape[1]
    g, seq, ctx = target.shape
    h = W1.shape[1]
    h2 = W2.shape[1]
    nout = Wo.shape[1]
    epc = _NW * _IB * _CH
    ep = ((e + epc - 1) // epc) * epc
    nchunkp = ep // _CH
    it = jnp.arange(ep - e, dtype=jnp.int32)
    src = jnp.concatenate([edge_index[0], it % n])
    dst2d = jnp.concatenate([edge_index[1],
                             n + (it % _PAD)]).reshape(nchunkp, _CH)

    deg2 = _make_deg_kernel(n, nchunkp)(dst2d)
    deg2 = deg2.reshape(2 * n, 1)

    bn = 2000
    nblk = n // bn
    hq = h // 2
    dinv, u1a, u1b = pl.pallas_call(
        _prep_body,
        grid=(nblk,),
        in_specs=[
            pl.BlockSpec((bn, 1), lambda i: (i, 0)),
            pl.BlockSpec((bn, 1), lambda i: (i + nblk, 0)),
            pl.BlockSpec((bn, d_in), lambda i: (i, 0)),
            pl.BlockSpec((d_in, hq), lambda i: (0, 0)),
            pl.BlockSpec((d_in, hq), lambda i: (0, 0)),
        ],
        out_specs=[
            pl.BlockSpec((bn, 1), lambda i: (i, 0)),
            pl.BlockSpec((bn, hq), lambda i: (i, 0)),
            pl.BlockSpec((bn, hq), lambda i: (i, 0)),
        ],
        out_shape=[
            jax.ShapeDtypeStruct((n, 1), jnp.float32),
            jax.ShapeDtypeStruct((n, hq), jnp.float32),
            jax.ShapeDtypeStruct((n, hq), jnp.float32),
        ],
    )(deg2, deg2, x, W1[:, :hq], W1[:, hq:])

    acc1a, acc1b = _make_scatter_col_kernel(n, hq, nchunkp)(u1a, u1b, src,
                                                            dst2d)

    hh = h2 // 2
    u2a, u2b = pl.pallas_call(
        _mid_body,
        grid=(nblk,),
        in_specs=[
            pl.BlockSpec((bn, hq), lambda i: (i, 0)),
            pl.BlockSpec((bn, hq), lambda i: (i, 0)),
            pl.BlockSpec((bn, 1), lambda i: (i, 0)),
            pl.BlockSpec((1, h), lambda i: (0, 0)),
            pl.BlockSpec((h, hh), lambda i: (0, 0)),
            pl.BlockSpec((h, hh), lambda i: (0, 0)),
        ],
        out_specs=[
            pl.BlockSpec((bn, hh), lambda i: (i, 0)),
            pl.BlockSpec((bn, hh), lambda i: (i, 0)),
        ],
        out_shape=[
            jax.ShapeDtypeStruct((n, hh), jnp.float32),
            jax.ShapeDtypeStruct((n, hh), jnp.float32),
        ],
    )(acc1a, acc1b, dinv, b1.reshape(1, h), W2[:, :hh], W2[:, hh:])

    acc2a, acc2b = _make_scatter_col_kernel(n, hh, nchunkp)(u2a, u2b, src,
                                                            dst2d)

    g0 = pl.pallas_call(
        functools.partial(_segmax_body, num_graphs=g),
        grid=(nblk,),
        in_specs=[
            pl.BlockSpec((bn, hh), lambda i: (i, 0)),
            pl.BlockSpec((bn, hh), lambda i: (i, 0)),
            pl.BlockSpec((bn, 1), lambda i: (i, 0)),
            pl.BlockSpec((1, h2), lambda i: (0, 0)),
            pl.BlockSpec((bn, 1), lambda i: (i, 0)),
        ],
        out_specs=pl.BlockSpec((g, h2), lambda i: (0, 0)),
        out_shape=jax.ShapeDtypeStruct((g, h2), jnp.float32),
    )(acc2a, acc2b, dinv, b2.reshape(1, h2), batch.reshape(n, 1))

    wc2 = jnp.transpose(Wc, (1, 2, 0)).reshape(ctx, 3 * h)
    wo_pad = jnp.zeros((h, 128), jnp.float32).at[:, :nout].set(Wo)
    bo_pad = jnp.zeros((1, 128), jnp.float32).at[:, :nout].set(bo)
    bg_rows = 8
    gblk = g // bg_rows
    out_pad = pl.pallas_call(
        functools.partial(_tail_body, bg_rows=bg_rows, seq=seq, ctx=ctx, h=h),
        grid=(gblk,),
        in_specs=[
            pl.BlockSpec((bg_rows, h2), lambda i: (i, 0)),
            pl.BlockSpec((bg_rows, seq, ctx), lambda i: (i, 0, 0)),
            pl.BlockSpec((h2, h), lambda i: (0, 0)),
            pl.BlockSpec((1, h), lambda i: (0, 0)),
            pl.BlockSpec((ctx, 3 * h), lambda i: (0, 0)),
            pl.BlockSpec((1, 1, h), lambda i: (0, 0, 0)),
            pl.BlockSpec((h, h), lambda i: (0, 0)),
            pl.BlockSpec((1, h), lambda i: (0, 0)),
            pl.BlockSpec((h2, h), lambda i: (0, 0)),
            pl.BlockSpec((1, h), lambda i: (0, 0)),
            pl.BlockSpec((h, 128), lambda i: (0, 0)),
            pl.BlockSpec((1, 128), lambda i: (0, 0)),
        ],
        out_specs=pl.BlockSpec((bg_rows, 128), lambda i: (i, 0)),
        out_shape=jax.ShapeDtypeStruct((g, 128), jnp.float32),
    )(g0, target, Wg, bg.reshape(1, h), wc2, bc.reshape(1, 1, h), Wxt,
      bxt.reshape(1, h), Wf, bf.reshape(1, h), wo_pad, bo_pad)

    return out_pad[:, :nout]
